# Initial kernel scaffold; baseline (speedup 1.0000x reference)
import jax
import jax.numpy as jnp
from jax.experimental import pallas as pl


def kernel(x, edge_index, eps1, W1, b1, W2, b2, eps2, W3, b3, W4, b4):
    raise NotImplementedError("write your pallas kernel here")


# speedup vs baseline: 3.9462x; 3.9462x over previous
import functools

import jax
import jax.numpy as jnp
from jax import lax
from jax.experimental import pallas as pl
from jax.experimental.pallas import tpu as pltpu
from jax.experimental.pallas import tpu_sc as plsc

_N = 10000
_E = 320000
_CHUNK = 128
_NC, _NS = 2, 16
_NW = _NC * _NS
_CPW = 80
_EPAD = _NW * _CPW * _CHUNK
_ACC_ROWS = 10112
_PAD_ROW = 10048
_ROWS_PER_TILE = _ACC_ROWS // _NS


def _sc_segment_sum(z, src2d, dst2d, zeros, d):
    mesh = plsc.VectorSubcoreMesh(core_axis_name="c", subcore_axis_name="s")

    @functools.partial(
        pl.kernel,
        out_type=jax.ShapeDtypeStruct((_NC * _ACC_ROWS, d), jnp.float32),
        mesh=mesh,
        scratch_types=[
            pltpu.VMEM((_CPW, _CHUNK), jnp.int32),
            pltpu.VMEM((_CPW, _CHUNK), jnp.int32),
            pltpu.VMEM((_CHUNK, d), jnp.float32),
            pltpu.VMEM_SHARED((_ACC_ROWS, d), jnp.float32),
            pltpu.SemaphoreType.DMA,
        ],
        compiler_params=pltpu.CompilerParams(use_tc_tiling_on_sc=False),
    )
    def k(z_hbm, src_hbm, dst_hbm, zeros_hbm, out_hbm, srcv, dstv, buf, acc, sem):
        c = lax.axis_index("c")
        s = lax.axis_index("s")
        wid = c * _NS + s
        rbase = s * _ROWS_PER_TILE
        pltpu.sync_copy(zeros_hbm.at[pl.ds(rbase, _ROWS_PER_TILE)],
                        acc.at[pl.ds(rbase, _ROWS_PER_TILE)])
        cbase = wid * _CPW
        pltpu.sync_copy(src_hbm.at[pl.ds(cbase, _CPW)], srcv)
        pltpu.sync_copy(dst_hbm.at[pl.ds(cbase, _CPW)], dstv)
        plsc.subcore_barrier()

        @pl.loop(0, _CPW)
        def _(j):
            pltpu.async_copy(z_hbm.at[srcv.at[j]], buf, sem).wait()
            pltpu.sync_copy(buf, acc.at[dstv.at[j]], add=True)

        plsc.subcore_barrier()
        pltpu.sync_copy(acc.at[pl.ds(rbase, _ROWS_PER_TILE)],
                        out_hbm.at[pl.ds(c * _ACC_ROWS + rbase, _ROWS_PER_TILE)])

    return k(z, src2d, dst2d, zeros)


_BLK = 1000


def _mlp1_body(scale_ref, x_ref, a0_ref, a1_ref, w1_ref, b1_ref, w2_ref,
               b2_ref, w3_ref, o_ref):
    u = x_ref[...] * scale_ref[0, 0] + a0_ref[...] + a1_ref[...]
    t = jnp.maximum(
        jnp.dot(u, w1_ref[...], preferred_element_type=jnp.float32) + b1_ref[...], 0.0)
    h = jnp.maximum(
        jnp.dot(t, w2_ref[...], preferred_element_type=jnp.float32) + b2_ref[...], 0.0)
    o_ref[...] = jnp.dot(h, w3_ref[...], preferred_element_type=jnp.float32)


def _tc_mlp1(x, a0, a1, scale, W1, b1, W2, b2, W3):
    grid = (_N // _BLK,)
    return pl.pallas_call(
        _mlp1_body,
        grid=grid,
        in_specs=[
            pl.BlockSpec(memory_space=pltpu.SMEM),
            pl.BlockSpec((_BLK, 128), lambda i: (i, 0)),
            pl.BlockSpec((_BLK, 128), lambda i: (i, 0)),
            pl.BlockSpec((_BLK, 128), lambda i: (i, 0)),
            pl.BlockSpec((128, 128), lambda i: (0, 0)),
            pl.BlockSpec((1, 128), lambda i: (0, 0)),
            pl.BlockSpec((128, 128), lambda i: (0, 0)),
            pl.BlockSpec((1, 128), lambda i: (0, 0)),
            pl.BlockSpec((128, 64), lambda i: (0, 0)),
        ],
        out_specs=pl.BlockSpec((_BLK, 64), lambda i: (i, 0)),
        out_shape=jax.ShapeDtypeStruct((_N, 64), jnp.float32),
    )(scale, x, a0, a1, W1, b1, W2, b2, W3)


def _mlp2_body(scale_ref, z_ref, c0_ref, c1_ref, b3_ref, w4_ref, b4_ref, o_ref):
    v = jnp.maximum(
        z_ref[...] * scale_ref[0, 0] + c0_ref[...] + c1_ref[...] + b3_ref[...], 0.0)
    o_ref[...] = jnp.dot(v, w4_ref[...], preferred_element_type=jnp.float32) + b4_ref[...]


def _tc_mlp2(z2, c0, c1, scale, b3, W4, b4):
    grid = (_N // _BLK,)
    return pl.pallas_call(
        _mlp2_body,
        grid=grid,
        in_specs=[
            pl.BlockSpec(memory_space=pltpu.SMEM),
            pl.BlockSpec((_BLK, 64), lambda i: (i, 0)),
            pl.BlockSpec((_BLK, 64), lambda i: (i, 0)),
            pl.BlockSpec((_BLK, 64), lambda i: (i, 0)),
            pl.BlockSpec((1, 64), lambda i: (0, 0)),
            pl.BlockSpec((64, 64), lambda i: (0, 0)),
            pl.BlockSpec((1, 64), lambda i: (0, 0)),
        ],
        out_specs=pl.BlockSpec((_BLK, 64), lambda i: (i, 0)),
        out_shape=jax.ShapeDtypeStruct((_N, 64), jnp.float32),
    )(scale, z2, c0, c1, b3, W4, b4)


def kernel(x, edge_index, eps1, W1, b1, W2, b2, eps2, W3, b3, W4, b4):
    src = edge_index[0].astype(jnp.int32)
    dst = edge_index[1].astype(jnp.int32)
    pad = _EPAD - _E
    src2d = jnp.concatenate([src, jnp.zeros((pad,), jnp.int32)]).reshape(-1, _CHUNK)
    dst2d = jnp.concatenate(
        [dst, jnp.full((pad,), _PAD_ROW, jnp.int32)]).reshape(-1, _CHUNK)
    zeros128 = jnp.zeros((_ACC_ROWS, 128), jnp.float32)
    zeros64 = jnp.zeros((_ACC_ROWS, 64), jnp.float32)

    agg1 = _sc_segment_sum(x, src2d, dst2d, zeros128, 128)
    a0 = agg1[0:_N]
    a1 = agg1[_ACC_ROWS:_ACC_ROWS + _N]
    scale1 = jnp.reshape(1.0 + eps1, (1, 1))
    z2 = _tc_mlp1(x, a0, a1, scale1, W1, b1.reshape(1, 128), W2,
                  b2.reshape(1, 128), W3)

    agg2 = _sc_segment_sum(z2, src2d, dst2d, zeros64, 64)
    c0 = agg2[0:_N]
    c1 = agg2[_ACC_ROWS:_ACC_ROWS + _N]
    scale2 = jnp.reshape(1.0 + eps2, (1, 1))
    return _tc_mlp2(z2, c0, c1, scale2, b3.reshape(1, 64), W4, b4.reshape(1, 64))


# speedup vs baseline: 4.1953x; 1.0631x over previous
import functools

import jax
import jax.numpy as jnp
from jax import lax
from jax.experimental import pallas as pl
from jax.experimental.pallas import tpu as pltpu
from jax.experimental.pallas import tpu_sc as plsc

_N = 10000
_E = 320000
_CHUNK = 128
_NC, _NS = 2, 16
_NW = _NC * _NS
_CPW = 80
_EPAD = _NW * _CPW * _CHUNK
_ACC_ROWS = 10112
_PAD_ROW = 10048
_ROWS_PER_TILE = _ACC_ROWS // _NS
_G = 16


def _sc_segment_sum(z, src2d, dst2d, zeros, d):
    mesh = plsc.VectorSubcoreMesh(core_axis_name="c", subcore_axis_name="s")

    @functools.partial(
        pl.kernel,
        out_type=jax.ShapeDtypeStruct((_NC * _ACC_ROWS, d), jnp.float32),
        mesh=mesh,
        scratch_types=[
            pltpu.VMEM((_G, _CHUNK), jnp.int32),
            pltpu.VMEM((_G, _CHUNK), jnp.int32),
            pltpu.VMEM((_CHUNK, d), jnp.float32),
            pltpu.VMEM((_CHUNK, d), jnp.float32),
            pltpu.VMEM_SHARED((_ACC_ROWS, d), jnp.float32),
            pltpu.SemaphoreType.DMA,
            pltpu.SemaphoreType.DMA,
        ],
        compiler_params=pltpu.CompilerParams(use_tc_tiling_on_sc=False),
    )
    def k(z_hbm, src_hbm, dst_hbm, zeros_hbm, out_hbm, srcv, dstv,
          buf0, buf1, acc, gsem, ssem):
        c = lax.axis_index("c")
        s = lax.axis_index("s")
        wid = c * _NS + s
        rbase = s * _ROWS_PER_TILE
        pltpu.sync_copy(zeros_hbm.at[pl.ds(rbase, _ROWS_PER_TILE)],
                        acc.at[pl.ds(rbase, _ROWS_PER_TILE)])
        cbase = wid * _CPW
        plsc.subcore_barrier()

        def _gather(j, buf):
            return pltpu.make_async_copy(z_hbm.at[srcv.at[j]], buf, gsem)

        def _scat(j, buf):
            return pltpu.make_async_copy(buf, acc.at[dstv.at[j]], ssem)

        @pl.loop(0, _CPW, step=_G)
        def _(g0):
            pltpu.sync_copy(src_hbm.at[pl.ds(cbase + g0, _G)], srcv)
            pltpu.sync_copy(dst_hbm.at[pl.ds(cbase + g0, _G)], dstv)
            _gather(0, buf0).start()

            @pl.loop(0, _G, step=2)
            def _(jj):
                _gather(jj, buf0).wait()
                _gather(jj + 1, buf1).start()
                _scat(jj, buf0).start(add=True)
                _gather(jj + 1, buf1).wait()
                _scat(jj + 1, buf1).start(add=True)
                _scat(jj, buf0).wait()

                @pl.when(jj + 2 < _G)
                def _():
                    _gather(jj + 2, buf0).start()

                _scat(jj + 1, buf1).wait()

        plsc.subcore_barrier()
        pltpu.sync_copy(acc.at[pl.ds(rbase, _ROWS_PER_TILE)],
                        out_hbm.at[pl.ds(c * _ACC_ROWS + rbase, _ROWS_PER_TILE)])

    return k(z, src2d, dst2d, zeros)


_BLK = 1000


def _mlp1_body(scale_ref, x_ref, a0_ref, a1_ref, w1_ref, b1_ref, w2_ref,
               b2_ref, w3_ref, o_ref):
    u = x_ref[...] * scale_ref[0, 0] + a0_ref[...] + a1_ref[...]
    t = jnp.maximum(
        jnp.dot(u, w1_ref[...], preferred_element_type=jnp.float32) + b1_ref[...], 0.0)
    h = jnp.maximum(
        jnp.dot(t, w2_ref[...], preferred_element_type=jnp.float32) + b2_ref[...], 0.0)
    o_ref[...] = jnp.dot(h, w3_ref[...], preferred_element_type=jnp.float32)


def _tc_mlp1(x, a0, a1, scale, W1, b1, W2, b2, W3):
    grid = (_N // _BLK,)
    return pl.pallas_call(
        _mlp1_body,
        grid=grid,
        in_specs=[
            pl.BlockSpec(memory_space=pltpu.SMEM),
            pl.BlockSpec((_BLK, 128), lambda i: (i, 0)),
            pl.BlockSpec((_BLK, 128), lambda i: (i, 0)),
            pl.BlockSpec((_BLK, 128), lambda i: (i, 0)),
            pl.BlockSpec((128, 128), lambda i: (0, 0)),
            pl.BlockSpec((1, 128), lambda i: (0, 0)),
            pl.BlockSpec((128, 128), lambda i: (0, 0)),
            pl.BlockSpec((1, 128), lambda i: (0, 0)),
            pl.BlockSpec((128, 64), lambda i: (0, 0)),
        ],
        out_specs=pl.BlockSpec((_BLK, 64), lambda i: (i, 0)),
        out_shape=jax.ShapeDtypeStruct((_N, 64), jnp.float32),
    )(scale, x, a0, a1, W1, b1, W2, b2, W3)


def _mlp2_body(scale_ref, z_ref, c0_ref, c1_ref, b3_ref, w4_ref, b4_ref, o_ref):
    v = jnp.maximum(
        z_ref[...] * scale_ref[0, 0] + c0_ref[...] + c1_ref[...] + b3_ref[...], 0.0)
    o_ref[...] = jnp.dot(v, w4_ref[...], preferred_element_type=jnp.float32) + b4_ref[...]


def _tc_mlp2(z2, c0, c1, scale, b3, W4, b4):
    grid = (_N // _BLK,)
    return pl.pallas_call(
        _mlp2_body,
        grid=grid,
        in_specs=[
            pl.BlockSpec(memory_space=pltpu.SMEM),
            pl.BlockSpec((_BLK, 64), lambda i: (i, 0)),
            pl.BlockSpec((_BLK, 64), lambda i: (i, 0)),
            pl.BlockSpec((_BLK, 64), lambda i: (i, 0)),
            pl.BlockSpec((1, 64), lambda i: (0, 0)),
            pl.BlockSpec((64, 64), lambda i: (0, 0)),
            pl.BlockSpec((1, 64), lambda i: (0, 0)),
        ],
        out_specs=pl.BlockSpec((_BLK, 64), lambda i: (i, 0)),
        out_shape=jax.ShapeDtypeStruct((_N, 64), jnp.float32),
    )(scale, z2, c0, c1, b3, W4, b4)


def kernel(x, edge_index, eps1, W1, b1, W2, b2, eps2, W3, b3, W4, b4):
    src = edge_index[0].astype(jnp.int32)
    dst = edge_index[1].astype(jnp.int32)
    pad = _EPAD - _E
    src2d = jnp.concatenate([src, jnp.zeros((pad,), jnp.int32)]).reshape(-1, _CHUNK)
    dst2d = jnp.concatenate(
        [dst, jnp.full((pad,), _PAD_ROW, jnp.int32)]).reshape(-1, _CHUNK)
    zeros128 = jnp.zeros((_ACC_ROWS, 128), jnp.float32)
    zeros64 = jnp.zeros((_ACC_ROWS, 64), jnp.float32)

    agg1 = _sc_segment_sum(x, src2d, dst2d, zeros128, 128)
    a0 = agg1[0:_N]
    a1 = agg1[_ACC_ROWS:_ACC_ROWS + _N]
    scale1 = jnp.reshape(1.0 + eps1, (1, 1))
    z2 = _tc_mlp1(x, a0, a1, scale1, W1, b1.reshape(1, 128), W2,
                  b2.reshape(1, 128), W3)

    agg2 = _sc_segment_sum(z2, src2d, dst2d, zeros64, 64)
    c0 = agg2[0:_N]
    c1 = agg2[_ACC_ROWS:_ACC_ROWS + _N]
    scale2 = jnp.reshape(1.0 + eps2, (1, 1))
    return _tc_mlp2(z2, c0, c1, scale2, b3.reshape(1, 64), W4, b4.reshape(1, 64))


# speedup vs baseline: 10.5917x; 2.5246x over previous
import functools

import numpy as np

import jax
import jax.numpy as jnp
from jax import lax
from jax.experimental import pallas as pl
from jax.experimental.pallas import tpu as pltpu
from jax.experimental.pallas import tpu_sc as plsc

_N = 10000
_E = 320000
_CHUNK = 128
_NC, _NS = 2, 16
_NW = _NC * _NS
_CPW = 80
_EPAD = _NW * _CPW * _CHUNK
_ACC_ROWS = 10112
_PAD_ROW = 10048
_ROWS_PER_TILE = _ACC_ROWS // _NS
_G = 16


def _sc_segment_sum(z, src2d, dst2d, zeros, d):
    mesh = plsc.VectorSubcoreMesh(core_axis_name="c", subcore_axis_name="s")

    @functools.partial(
        pl.kernel,
        out_type=jax.ShapeDtypeStruct((_NC * _ACC_ROWS, d), jnp.float32),
        mesh=mesh,
        scratch_types=[
            pltpu.VMEM((_G, _CHUNK), jnp.int32),
            pltpu.VMEM((_G, _CHUNK), jnp.int32),
            pltpu.VMEM((_CHUNK, d), jnp.float32),
            pltpu.VMEM((_CHUNK, d), jnp.float32),
            pltpu.VMEM_SHARED((_ACC_ROWS, d), jnp.float32),
            pltpu.SemaphoreType.DMA,
            pltpu.SemaphoreType.DMA,
        ],
        compiler_params=pltpu.CompilerParams(use_tc_tiling_on_sc=False),
    )
    def k(z_hbm, src_hbm, dst_hbm, zeros_hbm, out_hbm, srcv, dstv,
          buf0, buf1, acc, gsem, ssem):
        c = lax.axis_index("c")
        s = lax.axis_index("s")
        wid = c * _NS + s
        rbase = s * _ROWS_PER_TILE
        pltpu.sync_copy(zeros_hbm.at[pl.ds(rbase, _ROWS_PER_TILE)],
                        acc.at[pl.ds(rbase, _ROWS_PER_TILE)])
        cbase = wid * _CPW
        plsc.subcore_barrier()

        def _gather(j, buf):
            return pltpu.make_async_copy(z_hbm.at[srcv.at[j]], buf, gsem)

        def _scat(j, buf):
            return pltpu.make_async_copy(buf, acc.at[dstv.at[j]], ssem)

        @pl.loop(0, _CPW, step=_G)
        def _(g0):
            pltpu.sync_copy(src_hbm.at[pl.ds(cbase + g0, _G)], srcv)
            pltpu.sync_copy(dst_hbm.at[pl.ds(cbase + g0, _G)], dstv)
            _gather(0, buf0).start()

            @pl.loop(0, _G, step=2)
            def _(jj):
                _gather(jj, buf0).wait()
                _gather(jj + 1, buf1).start()
                _scat(jj, buf0).start(add=True)
                _gather(jj + 1, buf1).wait()
                _scat(jj + 1, buf1).start(add=True)
                _scat(jj, buf0).wait()

                @pl.when(jj + 2 < _G)
                def _():
                    _gather(jj + 2, buf0).start()

                _scat(jj + 1, buf1).wait()

        plsc.subcore_barrier()
        pltpu.sync_copy(acc.at[pl.ds(rbase, _ROWS_PER_TILE)],
                        out_hbm.at[pl.ds(c * _ACC_ROWS + rbase, _ROWS_PER_TILE)])

    return k(z, src2d, dst2d, zeros)


_BLK = 1000


def _mlp1_body(scale_ref, x_ref, a0_ref, a1_ref, w1_ref, b1_ref, w2_ref,
               b2_ref, w3_ref, o_ref):
    u = x_ref[...] * scale_ref[0, 0] + a0_ref[...] + a1_ref[...]
    t = jnp.maximum(
        jnp.dot(u, w1_ref[...], preferred_element_type=jnp.float32) + b1_ref[...], 0.0)
    h = jnp.maximum(
        jnp.dot(t, w2_ref[...], preferred_element_type=jnp.float32) + b2_ref[...], 0.0)
    o_ref[...] = jnp.dot(h, w3_ref[...], preferred_element_type=jnp.float32)


def _tc_mlp1(x, a0, a1, scale, W1, b1, W2, b2, W3):
    grid = (_N // _BLK,)
    return pl.pallas_call(
        _mlp1_body,
        grid=grid,
        in_specs=[
            pl.BlockSpec(memory_space=pltpu.SMEM),
            pl.BlockSpec((_BLK, 128), lambda i: (i, 0)),
            pl.BlockSpec((_BLK, 128), lambda i: (i, 0)),
            pl.BlockSpec((_BLK, 128), lambda i: (i, 0)),
            pl.BlockSpec((128, 128), lambda i: (0, 0)),
            pl.BlockSpec((1, 128), lambda i: (0, 0)),
            pl.BlockSpec((128, 128), lambda i: (0, 0)),
            pl.BlockSpec((1, 128), lambda i: (0, 0)),
            pl.BlockSpec((128, 64), lambda i: (0, 0)),
        ],
        out_specs=pl.BlockSpec((_BLK, 64), lambda i: (i, 0)),
        out_shape=jax.ShapeDtypeStruct((_N, 64), jnp.float32),
    )(scale, x, a0, a1, W1, b1, W2, b2, W3)


def _mlp2_body(scale_ref, z_ref, c0_ref, c1_ref, b3_ref, w4_ref, b4_ref, o_ref):
    v = jnp.maximum(
        z_ref[...] * scale_ref[0, 0] + c0_ref[...] + c1_ref[...] + b3_ref[...], 0.0)
    o_ref[...] = jnp.dot(v, w4_ref[...], preferred_element_type=jnp.float32) + b4_ref[...]


def _tc_mlp2(z2, c0, c1, scale, b3, W4, b4):
    grid = (_N // _BLK,)
    return pl.pallas_call(
        _mlp2_body,
        grid=grid,
        in_specs=[
            pl.BlockSpec(memory_space=pltpu.SMEM),
            pl.BlockSpec((_BLK, 64), lambda i: (i, 0)),
            pl.BlockSpec((_BLK, 64), lambda i: (i, 0)),
            pl.BlockSpec((_BLK, 64), lambda i: (i, 0)),
            pl.BlockSpec((1, 64), lambda i: (0, 0)),
            pl.BlockSpec((64, 64), lambda i: (0, 0)),
            pl.BlockSpec((1, 64), lambda i: (0, 0)),
        ],
        out_specs=pl.BlockSpec((_BLK, 64), lambda i: (i, 0)),
        out_shape=jax.ShapeDtypeStruct((_N, 64), jnp.float32),
    )(scale, z2, c0, c1, b3, W4, b4)


def kernel(x, edge_index, eps1, W1, b1, W2, b2, eps2, W3, b3, W4, b4):
    src = edge_index[0].astype(jnp.int32)
    dst = edge_index[1].astype(jnp.int32)
    pad = _EPAD - _E
    pad_src = jnp.asarray(np.arange(pad, dtype=np.int32) % _N)
    pad_dst = jnp.asarray(_N + (np.arange(pad, dtype=np.int32) % (_ACC_ROWS - _N)))
    src2d = jnp.concatenate([src, pad_src]).reshape(-1, _CHUNK)
    dst2d = jnp.concatenate([dst, pad_dst]).reshape(-1, _CHUNK)
    zeros128 = jnp.zeros((_ACC_ROWS, 128), jnp.float32)
    zeros64 = jnp.zeros((_ACC_ROWS, 64), jnp.float32)

    agg1 = _sc_segment_sum(x, src2d, dst2d, zeros128, 128)
    a0 = agg1[0:_N]
    a1 = agg1[_ACC_ROWS:_ACC_ROWS + _N]
    scale1 = jnp.reshape(1.0 + eps1, (1, 1))
    z2 = _tc_mlp1(x, a0, a1, scale1, W1, b1.reshape(1, 128), W2,
                  b2.reshape(1, 128), W3)

    agg2 = _sc_segment_sum(z2, src2d, dst2d, zeros64, 64)
    c0 = agg2[0:_N]
    c1 = agg2[_ACC_ROWS:_ACC_ROWS + _N]
    scale2 = jnp.reshape(1.0 + eps2, (1, 1))
    return _tc_mlp2(z2, c0, c1, scale2, b3.reshape(1, 64), W4, b4.reshape(1, 64))


# speedup vs baseline: 11.5578x; 1.0912x over previous
import functools

import numpy as np

import jax
import jax.numpy as jnp
from jax import lax
from jax.experimental import pallas as pl
from jax.experimental.pallas import tpu as pltpu
from jax.experimental.pallas import tpu_sc as plsc

_N = 10000
_E = 320000
_CHUNK = 128
_NC, _NS = 2, 16
_NW = _NC * _NS
_CPW = 80
_EPAD = _NW * _CPW * _CHUNK
_ACC_ROWS = 10112
_PAD_ROW = 10048
_ROWS_PER_TILE = _ACC_ROWS // _NS


def _sc_segment_sum(z, src2d, dst2d, zeros, d):
    mesh = plsc.VectorSubcoreMesh(core_axis_name="c", subcore_axis_name="s")
    depth = 2 if d == 128 else 8
    stage = 40 if d == 128 else _CPW

    @functools.partial(
        pl.kernel,
        out_type=jax.ShapeDtypeStruct((_NC * _ACC_ROWS, d), jnp.float32),
        mesh=mesh,
        scratch_types=[
            pltpu.VMEM((stage, _CHUNK), jnp.int32),
            pltpu.VMEM((stage, _CHUNK), jnp.int32),
            pltpu.VMEM((depth, _CHUNK, d), jnp.float32),
            pltpu.VMEM_SHARED((_ACC_ROWS, d), jnp.float32),
            pltpu.SemaphoreType.DMA((depth,)),
            pltpu.SemaphoreType.DMA((depth,)),
        ],
        compiler_params=pltpu.CompilerParams(use_tc_tiling_on_sc=False),
    )
    def k(z_hbm, src_hbm, dst_hbm, zeros_hbm, out_hbm, srcv, dstv,
          bufs, acc, gsem, ssem):
        c = lax.axis_index("c")
        s = lax.axis_index("s")
        wid = c * _NS + s
        rbase = s * _ROWS_PER_TILE
        cbase = wid * _CPW
        pltpu.sync_copy(zeros_hbm.at[pl.ds(rbase, _ROWS_PER_TILE)],
                        acc.at[pl.ds(rbase, _ROWS_PER_TILE)])
        plsc.subcore_barrier()

        def _gather(j, p):
            return pltpu.make_async_copy(z_hbm.at[srcv.at[j]], bufs.at[p],
                                         gsem.at[p])

        def _scat(j, p):
            return pltpu.make_async_copy(bufs.at[p], acc.at[dstv.at[j]],
                                         ssem.at[p])

        @pl.loop(0, _CPW, step=stage)
        def _(g0):
            pltpu.sync_copy(src_hbm.at[pl.ds(cbase + g0, stage)], srcv)
            pltpu.sync_copy(dst_hbm.at[pl.ds(cbase + g0, stage)], dstv)
            for p in range(depth):
                _gather(p, p).start()

            @pl.loop(0, stage, step=depth)
            def _(j0):
                for p in range(depth):
                    _gather(j0 + p, p).wait()
                    _scat(j0 + p, p).start(add=True)
                for p in range(depth):
                    _scat(j0 + p, p).wait()

                    @pl.when(j0 + p + depth < stage)
                    def _():
                        _gather(j0 + p + depth, p).start()

        plsc.subcore_barrier()
        pltpu.sync_copy(acc.at[pl.ds(rbase, _ROWS_PER_TILE)],
                        out_hbm.at[pl.ds(c * _ACC_ROWS + rbase, _ROWS_PER_TILE)])

    return k(z, src2d, dst2d, zeros)


_BLK = 1000


def _mlp1_body(scale_ref, x_ref, a0_ref, a1_ref, w1_ref, b1_ref, w2_ref,
               b2_ref, w3_ref, o_ref):
    u = x_ref[...] * scale_ref[0, 0] + a0_ref[...] + a1_ref[...]
    t = jnp.maximum(
        jnp.dot(u, w1_ref[...], preferred_element_type=jnp.float32) + b1_ref[...], 0.0)
    h = jnp.maximum(
        jnp.dot(t, w2_ref[...], preferred_element_type=jnp.float32) + b2_ref[...], 0.0)
    o_ref[...] = jnp.dot(h, w3_ref[...], preferred_element_type=jnp.float32)


def _tc_mlp1(x, a0, a1, scale, W1, b1, W2, b2, W3):
    grid = (_N // _BLK,)
    return pl.pallas_call(
        _mlp1_body,
        grid=grid,
        in_specs=[
            pl.BlockSpec(memory_space=pltpu.SMEM),
            pl.BlockSpec((_BLK, 128), lambda i: (i, 0)),
            pl.BlockSpec((_BLK, 128), lambda i: (i, 0)),
            pl.BlockSpec((_BLK, 128), lambda i: (i, 0)),
            pl.BlockSpec((128, 128), lambda i: (0, 0)),
            pl.BlockSpec((1, 128), lambda i: (0, 0)),
            pl.BlockSpec((128, 128), lambda i: (0, 0)),
            pl.BlockSpec((1, 128), lambda i: (0, 0)),
            pl.BlockSpec((128, 64), lambda i: (0, 0)),
        ],
        out_specs=pl.BlockSpec((_BLK, 64), lambda i: (i, 0)),
        out_shape=jax.ShapeDtypeStruct((_N, 64), jnp.float32),
    )(scale, x, a0, a1, W1, b1, W2, b2, W3)


def _mlp2_body(scale_ref, z_ref, c0_ref, c1_ref, b3_ref, w4_ref, b4_ref, o_ref):
    v = jnp.maximum(
        z_ref[...] * scale_ref[0, 0] + c0_ref[...] + c1_ref[...] + b3_ref[...], 0.0)
    o_ref[...] = jnp.dot(v, w4_ref[...], preferred_element_type=jnp.float32) + b4_ref[...]


def _tc_mlp2(z2, c0, c1, scale, b3, W4, b4):
    grid = (_N // _BLK,)
    return pl.pallas_call(
        _mlp2_body,
        grid=grid,
        in_specs=[
            pl.BlockSpec(memory_space=pltpu.SMEM),
            pl.BlockSpec((_BLK, 64), lambda i: (i, 0)),
            pl.BlockSpec((_BLK, 64), lambda i: (i, 0)),
            pl.BlockSpec((_BLK, 64), lambda i: (i, 0)),
            pl.BlockSpec((1, 64), lambda i: (0, 0)),
            pl.BlockSpec((64, 64), lambda i: (0, 0)),
            pl.BlockSpec((1, 64), lambda i: (0, 0)),
        ],
        out_specs=pl.BlockSpec((_BLK, 64), lambda i: (i, 0)),
        out_shape=jax.ShapeDtypeStruct((_N, 64), jnp.float32),
    )(scale, z2, c0, c1, b3, W4, b4)


def kernel(x, edge_index, eps1, W1, b1, W2, b2, eps2, W3, b3, W4, b4):
    src = edge_index[0].astype(jnp.int32)
    dst = edge_index[1].astype(jnp.int32)
    pad = _EPAD - _E
    pad_src = jnp.asarray(np.arange(pad, dtype=np.int32) % _N)
    pad_dst = jnp.asarray(_N + (np.arange(pad, dtype=np.int32) % (_ACC_ROWS - _N)))
    src2d = jnp.concatenate([src, pad_src]).reshape(-1, _CHUNK)
    dst2d = jnp.concatenate([dst, pad_dst]).reshape(-1, _CHUNK)
    zeros128 = jnp.zeros((_ACC_ROWS, 128), jnp.float32)
    zeros64 = jnp.zeros((_ACC_ROWS, 64), jnp.float32)

    agg1 = _sc_segment_sum(x, src2d, dst2d, zeros128, 128)
    a0 = agg1[0:_N]
    a1 = agg1[_ACC_ROWS:_ACC_ROWS + _N]
    scale1 = jnp.reshape(1.0 + eps1, (1, 1))
    z2 = _tc_mlp1(x, a0, a1, scale1, W1, b1.reshape(1, 128), W2,
                  b2.reshape(1, 128), W3)

    agg2 = _sc_segment_sum(z2, src2d, dst2d, zeros64, 64)
    c0 = agg2[0:_N]
    c1 = agg2[_ACC_ROWS:_ACC_ROWS + _N]
    scale2 = jnp.reshape(1.0 + eps2, (1, 1))
    return _tc_mlp2(z2, c0, c1, scale2, b3.reshape(1, 64), W4, b4.reshape(1, 64))


# speedup vs baseline: 12.0214x; 1.0401x over previous
import functools

import numpy as np

import jax
import jax.numpy as jnp
from jax import lax
from jax.experimental import pallas as pl
from jax.experimental.pallas import tpu as pltpu
from jax.experimental.pallas import tpu_sc as plsc

_N = 10000
_E = 320000
_CHUNK = 128
_NC, _NS = 2, 16
_NW = _NC * _NS
_CPW = 80
_EPAD = _NW * _CPW * _CHUNK
_ACC_ROWS = 10112
_PAD_ROW = 10048
_ROWS_PER_TILE = _ACC_ROWS // _NS


def _sc_segment_sum(z, src2d, dst2d, zeros, d):
    mesh = plsc.VectorSubcoreMesh(core_axis_name="c", subcore_axis_name="s")
    depth = 2 if d == 128 else 8
    stage = 40 if d == 128 else _CPW

    @functools.partial(
        pl.kernel,
        out_type=jax.ShapeDtypeStruct((_NC * _ACC_ROWS, d), jnp.float32),
        mesh=mesh,
        scratch_types=[
            pltpu.VMEM((stage, _CHUNK), jnp.int32),
            pltpu.VMEM((stage, _CHUNK), jnp.int32),
            pltpu.VMEM((depth, _CHUNK, d), jnp.float32),
            pltpu.VMEM_SHARED((_ACC_ROWS, d), jnp.float32),
            pltpu.SemaphoreType.DMA((depth,)),
            pltpu.SemaphoreType.DMA((depth,)),
        ],
        compiler_params=pltpu.CompilerParams(use_tc_tiling_on_sc=False),
    )
    def k(z_hbm, src_hbm, dst_hbm, zeros_hbm, out_hbm, srcv, dstv,
          bufs, acc, gsem, ssem):
        c = lax.axis_index("c")
        s = lax.axis_index("s")
        wid = c * _NS + s
        rbase = s * _ROWS_PER_TILE
        cbase = wid * _CPW
        pltpu.sync_copy(zeros_hbm.at[pl.ds(rbase, _ROWS_PER_TILE)],
                        acc.at[pl.ds(rbase, _ROWS_PER_TILE)])
        plsc.subcore_barrier()

        def _gather(j, p):
            return pltpu.make_async_copy(z_hbm.at[srcv.at[j]], bufs.at[p],
                                         gsem.at[p])

        def _scat(j, p):
            return pltpu.make_async_copy(bufs.at[p], acc.at[dstv.at[j]],
                                         ssem.at[p])

        @pl.loop(0, _CPW, step=stage)
        def _(g0):
            pltpu.sync_copy(src_hbm.at[pl.ds(cbase + g0, stage)], srcv)
            pltpu.sync_copy(dst_hbm.at[pl.ds(cbase + g0, stage)], dstv)
            for p in range(depth):
                _gather(p, p).start()

            @pl.loop(0, stage, step=depth)
            def _(j0):
                for p in range(depth):
                    _gather(j0 + p, p).wait()
                    _scat(j0 + p, p).start(add=True)
                for p in range(depth):
                    _scat(j0 + p, p).wait()

                    @pl.when(j0 + p + depth < stage)
                    def _():
                        _gather(j0 + p + depth, p).start()

        plsc.subcore_barrier()
        pltpu.sync_copy(acc.at[pl.ds(rbase, _ROWS_PER_TILE)],
                        out_hbm.at[pl.ds(c * _ACC_ROWS + rbase, _ROWS_PER_TILE)])

    return k(z, src2d, dst2d, zeros)


_CPW2 = 2 * _CPW


def _sc_segment_sum_colsplit(z_cat, src2d2, dst2d, zeros):
    mesh = plsc.VectorSubcoreMesh(core_axis_name="c", subcore_axis_name="s")
    depth = 8
    stage = 80

    @functools.partial(
        pl.kernel,
        out_type=jax.ShapeDtypeStruct((_NC * _ACC_ROWS, 64), jnp.float32),
        mesh=mesh,
        scratch_types=[
            pltpu.VMEM((stage, _CHUNK), jnp.int32),
            pltpu.VMEM((stage, _CHUNK), jnp.int32),
            pltpu.VMEM((depth, _CHUNK, 64), jnp.float32),
            pltpu.VMEM_SHARED((_ACC_ROWS, 64), jnp.float32),
            pltpu.SemaphoreType.DMA((depth,)),
            pltpu.SemaphoreType.DMA((depth,)),
        ],
        compiler_params=pltpu.CompilerParams(use_tc_tiling_on_sc=False),
    )
    def k(z_hbm, src_hbm, dst_hbm, zeros_hbm, out_hbm, srcv, dstv,
          bufs, acc, gsem, ssem):
        c = lax.axis_index("c")
        s = lax.axis_index("s")
        rbase = s * _ROWS_PER_TILE
        sbase = c * (_NW * _CPW) + s * _CPW2
        dbase = s * _CPW2
        pltpu.sync_copy(zeros_hbm.at[pl.ds(rbase, _ROWS_PER_TILE)],
                        acc.at[pl.ds(rbase, _ROWS_PER_TILE)])
        plsc.subcore_barrier()

        def _gather(j, p):
            return pltpu.make_async_copy(z_hbm.at[srcv.at[j]], bufs.at[p],
                                         gsem.at[p])

        def _scat(j, p):
            return pltpu.make_async_copy(bufs.at[p], acc.at[dstv.at[j]],
                                         ssem.at[p])

        @pl.loop(0, _CPW2, step=stage)
        def _(g0):
            pltpu.sync_copy(src_hbm.at[pl.ds(sbase + g0, stage)], srcv)
            pltpu.sync_copy(dst_hbm.at[pl.ds(dbase + g0, stage)], dstv)
            for p in range(depth):
                _gather(p, p).start()

            @pl.loop(0, stage, step=depth)
            def _(j0):
                for p in range(depth):
                    _gather(j0 + p, p).wait()
                    _scat(j0 + p, p).start(add=True)
                for p in range(depth):
                    _scat(j0 + p, p).wait()

                    @pl.when(j0 + p + depth < stage)
                    def _():
                        _gather(j0 + p + depth, p).start()

        plsc.subcore_barrier()
        pltpu.sync_copy(acc.at[pl.ds(rbase, _ROWS_PER_TILE)],
                        out_hbm.at[pl.ds(c * _ACC_ROWS + rbase, _ROWS_PER_TILE)])

    return k(z_cat, src2d2, dst2d, zeros)


_BLK = 1000


def _mlp1_body(scale_ref, x_ref, a0_ref, a1_ref, w1_ref, b1_ref, w2_ref,
               b2_ref, w3_ref, o_ref):
    agg = jnp.concatenate([a0_ref[...], a1_ref[...]], axis=1)
    u = x_ref[...] * scale_ref[0, 0] + agg
    t = jnp.maximum(
        jnp.dot(u, w1_ref[...], preferred_element_type=jnp.float32) + b1_ref[...], 0.0)
    h = jnp.maximum(
        jnp.dot(t, w2_ref[...], preferred_element_type=jnp.float32) + b2_ref[...], 0.0)
    o_ref[...] = jnp.dot(h, w3_ref[...], preferred_element_type=jnp.float32)


def _tc_mlp1(x, a0, a1, scale, W1, b1, W2, b2, W3):
    grid = (_N // _BLK,)
    return pl.pallas_call(
        _mlp1_body,
        grid=grid,
        in_specs=[
            pl.BlockSpec(memory_space=pltpu.SMEM),
            pl.BlockSpec((_BLK, 128), lambda i: (i, 0)),
            pl.BlockSpec((_BLK, 64), lambda i: (i, 0)),
            pl.BlockSpec((_BLK, 64), lambda i: (i, 0)),
            pl.BlockSpec((128, 128), lambda i: (0, 0)),
            pl.BlockSpec((1, 128), lambda i: (0, 0)),
            pl.BlockSpec((128, 128), lambda i: (0, 0)),
            pl.BlockSpec((1, 128), lambda i: (0, 0)),
            pl.BlockSpec((128, 64), lambda i: (0, 0)),
        ],
        out_specs=pl.BlockSpec((_BLK, 64), lambda i: (i, 0)),
        out_shape=jax.ShapeDtypeStruct((_N, 64), jnp.float32),
    )(scale, x, a0, a1, W1, b1, W2, b2, W3)


def _mlp2_body(scale_ref, z_ref, c0_ref, c1_ref, b3_ref, w4_ref, b4_ref, o_ref):
    v = jnp.maximum(
        z_ref[...] * scale_ref[0, 0] + c0_ref[...] + c1_ref[...] + b3_ref[...], 0.0)
    o_ref[...] = jnp.dot(v, w4_ref[...], preferred_element_type=jnp.float32) + b4_ref[...]


def _tc_mlp2(z2, c0, c1, scale, b3, W4, b4):
    grid = (_N // _BLK,)
    return pl.pallas_call(
        _mlp2_body,
        grid=grid,
        in_specs=[
            pl.BlockSpec(memory_space=pltpu.SMEM),
            pl.BlockSpec((_BLK, 64), lambda i: (i, 0)),
            pl.BlockSpec((_BLK, 64), lambda i: (i, 0)),
            pl.BlockSpec((_BLK, 64), lambda i: (i, 0)),
            pl.BlockSpec((1, 64), lambda i: (0, 0)),
            pl.BlockSpec((64, 64), lambda i: (0, 0)),
            pl.BlockSpec((1, 64), lambda i: (0, 0)),
        ],
        out_specs=pl.BlockSpec((_BLK, 64), lambda i: (i, 0)),
        out_shape=jax.ShapeDtypeStruct((_N, 64), jnp.float32),
    )(scale, z2, c0, c1, b3, W4, b4)


def kernel(x, edge_index, eps1, W1, b1, W2, b2, eps2, W3, b3, W4, b4):
    src = edge_index[0].astype(jnp.int32)
    dst = edge_index[1].astype(jnp.int32)
    pad = _EPAD - _E
    pad_src = jnp.asarray(np.arange(pad, dtype=np.int32) % _N)
    pad_dst = jnp.asarray(_N + (np.arange(pad, dtype=np.int32) % (_ACC_ROWS - _N)))
    src2d = jnp.concatenate([src, pad_src]).reshape(-1, _CHUNK)
    dst2d = jnp.concatenate([dst, pad_dst]).reshape(-1, _CHUNK)
    zeros64 = jnp.zeros((_ACC_ROWS, 64), jnp.float32)

    x_cat = jnp.concatenate([x[:, :64], x[:, 64:]], axis=0)
    src2d2 = jnp.concatenate([src2d, src2d + _N], axis=0)
    agg1 = _sc_segment_sum_colsplit(x_cat, src2d2, dst2d, zeros64)
    a0 = agg1[0:_N]
    a1 = agg1[_ACC_ROWS:_ACC_ROWS + _N]
    scale1 = jnp.reshape(1.0 + eps1, (1, 1))
    z2 = _tc_mlp1(x, a0, a1, scale1, W1, b1.reshape(1, 128), W2,
                  b2.reshape(1, 128), W3)

    agg2 = _sc_segment_sum(z2, src2d, dst2d, zeros64, 64)
    c0 = agg2[0:_N]
    c1 = agg2[_ACC_ROWS:_ACC_ROWS + _N]
    scale2 = jnp.reshape(1.0 + eps2, (1, 1))
    return _tc_mlp2(z2, c0, c1, scale2, b3.reshape(1, 64), W4, b4.reshape(1, 64))


# speedup vs baseline: 15.1800x; 1.2627x over previous
import functools

import numpy as np

import jax
import jax.numpy as jnp
from jax import lax
from jax.experimental import pallas as pl
from jax.experimental.pallas import tpu as pltpu
from jax.experimental.pallas import tpu_sc as plsc

_N = 10000
_E = 320000
_CHUNK = 128
_NC, _NS = 2, 16
_NW = _NC * _NS
_CPW = 80
_EPAD = _NW * _CPW * _CHUNK
_ACC_ROWS = 10240
_ROWS_PER_TILE = _ACC_ROWS // _NS


def _sc_segment_sum(z, src2d, dst2d, zeros, d):
    mesh = plsc.VectorSubcoreMesh(core_axis_name="c", subcore_axis_name="s")
    depth = 2 if d == 128 else 8
    stage = 40 if d == 128 else _CPW

    @functools.partial(
        pl.kernel,
        out_type=jax.ShapeDtypeStruct((_NC * _ACC_ROWS, d), jnp.bfloat16),
        mesh=mesh,
        scratch_types=[
            pltpu.VMEM((stage, _CHUNK), jnp.int32),
            pltpu.VMEM((stage, _CHUNK), jnp.int32),
            pltpu.VMEM((depth, _CHUNK, d), jnp.bfloat16),
            pltpu.VMEM_SHARED((_ACC_ROWS, d), jnp.bfloat16),
            pltpu.SemaphoreType.DMA((depth,)),
            pltpu.SemaphoreType.DMA((depth,)),
        ],
        compiler_params=pltpu.CompilerParams(use_tc_tiling_on_sc=False),
    )
    def k(z_hbm, src_hbm, dst_hbm, zeros_hbm, out_hbm, srcv, dstv,
          bufs, acc, gsem, ssem):
        c = lax.axis_index("c")
        s = lax.axis_index("s")
        wid = c * _NS + s
        rbase = s * _ROWS_PER_TILE
        cbase = wid * _CPW
        pltpu.sync_copy(zeros_hbm.at[pl.ds(rbase, _ROWS_PER_TILE)],
                        acc.at[pl.ds(rbase, _ROWS_PER_TILE)])
        plsc.subcore_barrier()

        def _gather(j, p):
            return pltpu.make_async_copy(z_hbm.at[srcv.at[j]], bufs.at[p],
                                         gsem.at[p])

        def _scat(j, p):
            return pltpu.make_async_copy(bufs.at[p], acc.at[dstv.at[j]],
                                         ssem.at[p])

        @pl.loop(0, _CPW, step=stage)
        def _(g0):
            pltpu.sync_copy(src_hbm.at[pl.ds(cbase + g0, stage)], srcv)
            pltpu.sync_copy(dst_hbm.at[pl.ds(cbase + g0, stage)], dstv)
            for p in range(depth):
                _gather(p, p).start()

            @pl.loop(0, stage, step=depth)
            def _(j0):
                for p in range(depth):
                    _gather(j0 + p, p).wait()
                    _scat(j0 + p, p).start(add=True)
                for p in range(depth):
                    _scat(j0 + p, p).wait()

                    @pl.when(j0 + p + depth < stage)
                    def _():
                        _gather(j0 + p + depth, p).start()

        plsc.subcore_barrier()
        pltpu.sync_copy(acc.at[pl.ds(rbase, _ROWS_PER_TILE)],
                        out_hbm.at[pl.ds(c * _ACC_ROWS + rbase, _ROWS_PER_TILE)])

    return k(z, src2d, dst2d, zeros)


_CPW2 = 2 * _CPW


def _sc_segment_sum_colsplit(z_cat, src2d2, dst2d, zeros):
    mesh = plsc.VectorSubcoreMesh(core_axis_name="c", subcore_axis_name="s")
    depth = 8
    stage = 80

    @functools.partial(
        pl.kernel,
        out_type=jax.ShapeDtypeStruct((_NC * _ACC_ROWS, 64), jnp.bfloat16),
        mesh=mesh,
        scratch_types=[
            pltpu.VMEM((stage, _CHUNK), jnp.int32),
            pltpu.VMEM((stage, _CHUNK), jnp.int32),
            pltpu.VMEM((depth, _CHUNK, 64), jnp.bfloat16),
            pltpu.VMEM_SHARED((_ACC_ROWS, 64), jnp.bfloat16),
            pltpu.SemaphoreType.DMA((depth,)),
            pltpu.SemaphoreType.DMA((depth,)),
        ],
        compiler_params=pltpu.CompilerParams(use_tc_tiling_on_sc=False),
    )
    def k(z_hbm, src_hbm, dst_hbm, zeros_hbm, out_hbm, srcv, dstv,
          bufs, acc, gsem, ssem):
        c = lax.axis_index("c")
        s = lax.axis_index("s")
        rbase = s * _ROWS_PER_TILE
        sbase = c * (_NW * _CPW) + s * _CPW2
        dbase = s * _CPW2
        pltpu.sync_copy(zeros_hbm.at[pl.ds(rbase, _ROWS_PER_TILE)],
                        acc.at[pl.ds(rbase, _ROWS_PER_TILE)])
        plsc.subcore_barrier()

        def _gather(j, p):
            return pltpu.make_async_copy(z_hbm.at[srcv.at[j]], bufs.at[p],
                                         gsem.at[p])

        def _scat(j, p):
            return pltpu.make_async_copy(bufs.at[p], acc.at[dstv.at[j]],
                                         ssem.at[p])

        @pl.loop(0, _CPW2, step=stage)
        def _(g0):
            pltpu.sync_copy(src_hbm.at[pl.ds(sbase + g0, stage)], srcv)
            pltpu.sync_copy(dst_hbm.at[pl.ds(dbase + g0, stage)], dstv)
            for p in range(depth):
                _gather(p, p).start()

            @pl.loop(0, stage, step=depth)
            def _(j0):
                for p in range(depth):
                    _gather(j0 + p, p).wait()
                    _scat(j0 + p, p).start(add=True)
                for p in range(depth):
                    _scat(j0 + p, p).wait()

                    @pl.when(j0 + p + depth < stage)
                    def _():
                        _gather(j0 + p + depth, p).start()

        plsc.subcore_barrier()
        pltpu.sync_copy(acc.at[pl.ds(rbase, _ROWS_PER_TILE)],
                        out_hbm.at[pl.ds(c * _ACC_ROWS + rbase, _ROWS_PER_TILE)])

    return k(z_cat, src2d2, dst2d, zeros)


_BLK = 1000


def _mlp1_body(scale_ref, x_ref, a0_ref, a1_ref, w1_ref, b1_ref, w2_ref,
               b2_ref, w3_ref, o_ref):
    agg = jnp.concatenate([a0_ref[...], a1_ref[...]], axis=1).astype(jnp.float32)
    u = x_ref[...] * scale_ref[0, 0] + agg
    t = jnp.maximum(
        jnp.dot(u, w1_ref[...], preferred_element_type=jnp.float32) + b1_ref[...], 0.0)
    h = jnp.maximum(
        jnp.dot(t, w2_ref[...], preferred_element_type=jnp.float32) + b2_ref[...], 0.0)
    o_ref[...] = jnp.dot(
        h, w3_ref[...], preferred_element_type=jnp.float32).astype(jnp.bfloat16)


def _tc_mlp1(x, a0, a1, scale, W1, b1, W2, b2, W3):
    grid = (_N // _BLK,)
    return pl.pallas_call(
        _mlp1_body,
        grid=grid,
        in_specs=[
            pl.BlockSpec(memory_space=pltpu.SMEM),
            pl.BlockSpec((_BLK, 128), lambda i: (i, 0)),
            pl.BlockSpec((_BLK, 64), lambda i: (i, 0)),
            pl.BlockSpec((_BLK, 64), lambda i: (i, 0)),
            pl.BlockSpec((128, 128), lambda i: (0, 0)),
            pl.BlockSpec((1, 128), lambda i: (0, 0)),
            pl.BlockSpec((128, 128), lambda i: (0, 0)),
            pl.BlockSpec((1, 128), lambda i: (0, 0)),
            pl.BlockSpec((128, 64), lambda i: (0, 0)),
        ],
        out_specs=pl.BlockSpec((_BLK, 64), lambda i: (i, 0)),
        out_shape=jax.ShapeDtypeStruct((_N, 64), jnp.bfloat16),
    )(scale, x, a0, a1, W1, b1, W2, b2, W3)


def _mlp2_body(scale_ref, z_ref, c0_ref, c1_ref, b3_ref, w4_ref, b4_ref, o_ref):
    v = jnp.maximum(
        z_ref[...].astype(jnp.float32) * scale_ref[0, 0]
        + c0_ref[...].astype(jnp.float32) + c1_ref[...].astype(jnp.float32)
        + b3_ref[...], 0.0)
    o_ref[...] = jnp.dot(v, w4_ref[...], preferred_element_type=jnp.float32) + b4_ref[...]


def _tc_mlp2(z2, c0, c1, scale, b3, W4, b4):
    grid = (_N // _BLK,)
    return pl.pallas_call(
        _mlp2_body,
        grid=grid,
        in_specs=[
            pl.BlockSpec(memory_space=pltpu.SMEM),
            pl.BlockSpec((_BLK, 64), lambda i: (i, 0)),
            pl.BlockSpec((_BLK, 64), lambda i: (i, 0)),
            pl.BlockSpec((_BLK, 64), lambda i: (i, 0)),
            pl.BlockSpec((1, 64), lambda i: (0, 0)),
            pl.BlockSpec((64, 64), lambda i: (0, 0)),
            pl.BlockSpec((1, 64), lambda i: (0, 0)),
        ],
        out_specs=pl.BlockSpec((_BLK, 64), lambda i: (i, 0)),
        out_shape=jax.ShapeDtypeStruct((_N, 64), jnp.float32),
    )(scale, z2, c0, c1, b3, W4, b4)


def kernel(x, edge_index, eps1, W1, b1, W2, b2, eps2, W3, b3, W4, b4):
    src = edge_index[0].astype(jnp.int32)
    dst = edge_index[1].astype(jnp.int32)
    pad = _EPAD - _E
    pad_src = jnp.asarray(np.arange(pad, dtype=np.int32) % _N)
    pad_dst = jnp.asarray(_N + (np.arange(pad, dtype=np.int32) % (_ACC_ROWS - _N)))
    src2d = jnp.concatenate([src, pad_src]).reshape(-1, _CHUNK)
    dst2d = jnp.concatenate([dst, pad_dst]).reshape(-1, _CHUNK)
    zeros64 = jnp.zeros((_ACC_ROWS, 64), jnp.bfloat16)

    x_cat = jnp.concatenate([x[:, :64], x[:, 64:]], axis=0).astype(jnp.bfloat16)
    src2d2 = jnp.concatenate([src2d, src2d + _N], axis=0)
    agg1 = _sc_segment_sum_colsplit(x_cat, src2d2, dst2d, zeros64)
    a0 = agg1[0:_N]
    a1 = agg1[_ACC_ROWS:_ACC_ROWS + _N]
    scale1 = jnp.reshape(1.0 + eps1, (1, 1))
    z2 = _tc_mlp1(x, a0, a1, scale1, W1, b1.reshape(1, 128), W2,
                  b2.reshape(1, 128), W3)

    agg2 = _sc_segment_sum(z2, src2d, dst2d, zeros64, 64)
    c0 = agg2[0:_N]
    c1 = agg2[_ACC_ROWS:_ACC_ROWS + _N]
    scale2 = jnp.reshape(1.0 + eps2, (1, 1))
    return _tc_mlp2(z2, c0, c1, scale2, b3.reshape(1, 64), W4, b4.reshape(1, 64))
